# Initial kernel scaffold; baseline (speedup 1.0000x reference)
import jax
import jax.numpy as jnp
from jax.experimental import pallas as pl


def kernel(score, target):
    raise NotImplementedError("write your pallas kernel here")


# speedup vs baseline: 33.1102x; 33.1102x over previous
import jax
import jax.numpy as jnp
from jax.experimental import pallas as pl
from jax.experimental.pallas import tpu as pltpu

_IGNORE = 255
_THRESH_BITS = 0x3F19999A
_INF_BITS = 0x7F800000
_MIN_KEPT = 256

_B, _C, _H, _W = 8, 19, 512, 512
_BH = 128
_N = _B * _H * _W


def _pass1_body(score_ref, tgt_ref, loss_ref, pred_ref):
    t = tgt_ref[0]
    m = score_ref[0, 0]
    for c in range(1, _C):
        m = jnp.maximum(m, score_ref[0, c])
    s = jnp.zeros_like(m)
    xt = jnp.zeros_like(m)
    for c in range(_C):
        xc = score_ref[0, c]
        s = s + jnp.exp(xc - m)
        xt = jnp.where(t == c, xc, xt)
    logz = m + jnp.log(s)
    mask = t != _IGNORE
    loss_ref[0] = jnp.where(mask, logz - xt, 0.0)
    pred_ref[0] = jnp.where(mask, jnp.exp(xt - logz), jnp.inf)


def _pass1(score, target):
    nh = _H // _BH
    return pl.pallas_call(
        _pass1_body,
        grid=(_B, nh),
        in_specs=[
            pl.BlockSpec((1, _C, _BH, _W), lambda b, h: (b, 0, h, 0)),
            pl.BlockSpec((1, _BH, _W), lambda b, h: (b, h, 0)),
        ],
        out_specs=[
            pl.BlockSpec((1, _BH, _W), lambda b, h: (b, h, 0)),
            pl.BlockSpec((1, _BH, _W), lambda b, h: (b, h, 0)),
        ],
        out_shape=[
            jax.ShapeDtypeStruct((_B, _H, _W), jnp.float32),
            jax.ShapeDtypeStruct((_B, _H, _W), jnp.float32),
        ],
    )(score, target)


def _pass2_body(pred_ref, loss_ref, out_ref, thr_ref):
    p = pred_ref[...]
    l = loss_ref[...]
    bits = jax.lax.bitcast_convert_type(p, jnp.int32)
    n = jnp.sum(jnp.where(bits < _INF_BITS, 1.0, 0.0)).astype(jnp.int32)
    k = jnp.minimum(jnp.int32(_MIN_KEPT), n - 1)
    c06 = jnp.sum(jnp.where(bits < _THRESH_BITS, 1.0, 0.0)).astype(jnp.int32)
    thr_ref[0] = jnp.int32(_THRESH_BITS)

    @pl.when(c06 < k + 1)
    def _bisect():
        def body(_, lohi):
            lo, hi = lohi
            mid = jax.lax.div(lo + hi, jnp.int32(2))
            cnt = jnp.sum(jnp.where(bits <= mid, 1.0, 0.0)).astype(jnp.int32)
            good = cnt >= k + 1
            return (jnp.where(good, lo, mid + 1), jnp.where(good, mid, hi))

        lo, _hi = jax.lax.fori_loop(
            0, 31, body, (jnp.int32(0), jnp.int32(0x40000000))
        )
        thr_ref[0] = jnp.maximum(lo, jnp.int32(_THRESH_BITS))

    thr = thr_ref[0]
    keep = bits < thr
    cnt = jnp.sum(jnp.where(keep, 1.0, 0.0))
    s = jnp.sum(jnp.where(keep, l, 0.0))
    out_ref[0] = s / jnp.maximum(cnt, 1.0)


def _pass2(pred2, loss2):
    rows, cols = pred2.shape
    return pl.pallas_call(
        _pass2_body,
        in_specs=[
            pl.BlockSpec((rows, cols), lambda: (0, 0)),
            pl.BlockSpec((rows, cols), lambda: (0, 0)),
        ],
        out_specs=pl.BlockSpec(memory_space=pltpu.SMEM),
        out_shape=jax.ShapeDtypeStruct((1,), jnp.float32),
        scratch_shapes=[pltpu.SMEM((1,), jnp.int32)],
    )(pred2, loss2)


def kernel(score, target):
    loss_arr, pred_arr = _pass1(score, target)
    pred2 = pred_arr.reshape(2048, _N // 2048)
    loss2 = loss_arr.reshape(2048, _N // 2048)
    ohem = _pass2(pred2, loss2)[0]
    return (ohem, ohem, ohem - ohem)
